# baseline (speedup 1.0000x reference)
import functools

import jax
import jax.numpy as jnp
from jax import lax
from jax.experimental import pallas as pl
from jax.experimental.pallas import tpu as pltpu
from jax.experimental.pallas import tpu_sc as plsc

N = 10000
E = 320000
D = 128

NC = 2
NS = 16
NW = NC * NS

CH = 80
EPW = E // NW
NCHUNK = EPW // CH
SSTR = 624
LSTR = N - (NS - 1) * SSTR

assert NCHUNK * CH == EPW and NCHUNK % 2 == 1 and (NS - 1) * SSTR + LSTR == N

_mesh = plsc.VectorSubcoreMesh(
    core_axis_name="c", subcore_axis_name="s", num_cores=NC, num_subcores=NS
)


@functools.partial(
    pl.kernel,
    out_type=jax.ShapeDtypeStruct((NC, N, D), jnp.float32),
    mesh=_mesh,
    scratch_types=[
        pltpu.VMEM((EPW,), jnp.int32),
        [pltpu.VMEM((CH,), jnp.int32) for _ in range(3)],
        [pltpu.VMEM((CH, D), jnp.float32) for _ in range(3)],
        pltpu.VMEM_SHARED((N, D), jnp.float32),
        [pltpu.SemaphoreType.DMA for _ in range(3)],
        [pltpu.SemaphoreType.DMA for _ in range(3)],
        [pltpu.SemaphoreType.DMA for _ in range(3)],
    ],
)
def _sc_segment_sum(h_hbm, src_hbm, dst_hbm, zeros_hbm, out_hbm,
                    src_v, dst_bufs, row_bufs, agg_sh, gsems, dsems, ssems):
    c = lax.axis_index("c")
    s = lax.axis_index("s")
    wid = c * NS + s
    ebase = pl.multiple_of(wid * EPW, 8)

    pltpu.sync_copy(src_hbm.at[pl.ds(ebase, EPW)], src_v)

    @pl.when(s < NS - 1)
    def _():
        stripe = pl.ds(pl.multiple_of(s * SSTR, 8), SSTR)
        pltpu.sync_copy(zeros_hbm.at[stripe], agg_sh.at[stripe])

    @pl.when(s == NS - 1)
    def _():
        stripe = pl.ds((NS - 1) * SSTR, LSTR)
        pltpu.sync_copy(zeros_hbm.at[stripe], agg_sh.at[stripe])

    plsc.subcore_barrier()

    def start_g(j, b):
        pltpu.async_copy(h_hbm.at[src_v.at[pl.ds(j * CH, CH)]], row_bufs[b],
                         gsems[b])
        pltpu.async_copy(
            dst_hbm.at[pl.ds(pl.multiple_of(ebase + j * CH, 8), CH)],
            dst_bufs[b], dsems[b])

    def wait_g(b):
        pltpu.make_async_copy(h_hbm.at[src_v.at[pl.ds(0, CH)]], row_bufs[b],
                              gsems[b]).wait()
        pltpu.make_async_copy(dst_hbm.at[pl.ds(0, CH)], dst_bufs[b],
                              dsems[b]).wait()

    def start_s(b):
        pltpu.async_copy(row_bufs[b], agg_sh.at[dst_bufs[b]], ssems[b],
                         add=True)

    def wait_s(b):
        pltpu.make_async_copy(row_bufs[b], agg_sh.at[dst_bufs[b]],
                              ssems[b]).wait()

    start_g(0, 0)
    start_g(1, 1)
    wait_g(0); start_s(0); start_g(2, 2)
    wait_g(1); start_s(1); wait_s(0); start_g(3, 0)
    wait_g(2); start_s(2); wait_s(1); start_g(4, 1)

    @pl.loop(3, NCHUNK - 2, step=3)
    def _(j):
        wait_g(0); start_s(0); wait_s(2); start_g(j + 2, 2)
        wait_g(1); start_s(1); wait_s(0); start_g(j + 3, 0)
        wait_g(2); start_s(2); wait_s(1); start_g(j + 4, 1)

    wait_g(0); start_s(0); wait_s(2)
    wait_g(1); start_s(1); wait_s(0)
    wait_s(1)

    plsc.subcore_barrier()

    @pl.when(s < NS - 1)
    def _():
        stripe = pl.ds(pl.multiple_of(s * SSTR, 8), SSTR)
        pltpu.sync_copy(agg_sh.at[stripe], out_hbm.at[c, stripe])

    @pl.when(s == NS - 1)
    def _():
        stripe = pl.ds((NS - 1) * SSTR, LSTR)
        pltpu.sync_copy(agg_sh.at[stripe], out_hbm.at[c, stripe])


_BLK = 5000
_F32 = jnp.float32


def _dot(a, b):
    return jnp.dot(a, b, preferred_element_type=_F32,
                   precision=lax.Precision.HIGHEST)


def _pre_body(x_ref, wpt_ref, bp_ref, wst_ref, br_ref, xres_ref, hw_ref):
    x = x_ref[...]
    xres_ref[...] = _dot(x, wpt_ref[...]) + bp_ref[...]
    hw_ref[...] = _dot(x, wst_ref[...]) + br_ref[...]


def _pre(x, wpt, bp, wst, br):
    return pl.pallas_call(
        _pre_body,
        out_shape=(jax.ShapeDtypeStruct((N, D), _F32),
                   jax.ShapeDtypeStruct((N, D), _F32)),
        grid=(N // _BLK,),
        in_specs=[
            pl.BlockSpec((_BLK, D), lambda i: (i, 0)),
            pl.BlockSpec((D, D), lambda i: (0, 0)),
            pl.BlockSpec((1, D), lambda i: (0, 0)),
            pl.BlockSpec((D, D), lambda i: (0, 0)),
            pl.BlockSpec((1, D), lambda i: (0, 0)),
        ],
        out_specs=(pl.BlockSpec((_BLK, D), lambda i: (i, 0)),
                   pl.BlockSpec((_BLK, D), lambda i: (i, 0))),
    )(x, wpt, bp.reshape(1, D), wst, br.reshape(1, D))


def _layer_body(agg_a_ref, agg_b_ref, wrt_ref, hw_ref, add_ref,
                wst_ref, brn_ref, h_ref, hwn_ref):
    h = jnp.maximum(_dot(agg_a_ref[0] + agg_b_ref[0], wrt_ref[...])
                    + hw_ref[...], 0.0) + add_ref[...]
    h_ref[...] = h
    hwn_ref[...] = _dot(h, wst_ref[...]) + brn_ref[...]


def _layer(agg, wrt, hw, add, wst_next, br_next):
    return pl.pallas_call(
        _layer_body,
        out_shape=(jax.ShapeDtypeStruct((N, D), _F32),
                   jax.ShapeDtypeStruct((N, D), _F32)),
        grid=(N // _BLK,),
        in_specs=[
            pl.BlockSpec((1, _BLK, D), lambda i: (0, i, 0)),
            pl.BlockSpec((1, _BLK, D), lambda i: (1, i, 0)),
            pl.BlockSpec((D, D), lambda i: (0, 0)),
            pl.BlockSpec((_BLK, D), lambda i: (i, 0)),
            pl.BlockSpec((_BLK, D), lambda i: (i, 0)),
            pl.BlockSpec((D, D), lambda i: (0, 0)),
            pl.BlockSpec((1, D), lambda i: (0, 0)),
        ],
        out_specs=(pl.BlockSpec((_BLK, D), lambda i: (i, 0)),
                   pl.BlockSpec((_BLK, D), lambda i: (i, 0))),
    )(agg, agg, wrt, hw, add, wst_next, br_next.reshape(1, D))


def _final_body(agg_a_ref, agg_b_ref, wrt_ref, hw_ref, o_ref):
    o_ref[...] = _dot(agg_a_ref[0] + agg_b_ref[0], wrt_ref[...]) + hw_ref[...]


def _final(agg, wrt, hw):
    return pl.pallas_call(
        _final_body,
        out_shape=jax.ShapeDtypeStruct((N, D), _F32),
        grid=(N // _BLK,),
        in_specs=[
            pl.BlockSpec((1, _BLK, D), lambda i: (0, i, 0)),
            pl.BlockSpec((1, _BLK, D), lambda i: (1, i, 0)),
            pl.BlockSpec((D, D), lambda i: (0, 0)),
            pl.BlockSpec((_BLK, D), lambda i: (i, 0)),
        ],
        out_specs=pl.BlockSpec((_BLK, D), lambda i: (i, 0)),
    )(agg, agg, wrt, hw)


def kernel(x, edge_index, Wp, bp, Wr0, br0, Ws0, Wr1, br1, Ws1,
           Wr2, br2, Ws2, Wr3, br3, Ws3):
    edge_index = edge_index.astype(jnp.int32)
    src_r = edge_index[0]
    dst_r = edge_index[1]
    zeros = jnp.zeros((N, D), _F32)

    x_res, hw = _pre(x, Wp.T, bp, Ws0.T, br0)

    h = x
    agg = _sc_segment_sum(h, src_r, dst_r, zeros)
    h, hw = _layer(agg, Wr0.T, hw, x_res, Ws1.T, br1)

    agg = _sc_segment_sum(h, src_r, dst_r, zeros)
    h, hw = _layer(agg, Wr1.T, hw, h, Ws2.T, br2)

    agg = _sc_segment_sum(h, src_r, dst_r, zeros)
    h, hw = _layer(agg, Wr2.T, hw, h, Ws3.T, br3)

    agg = _sc_segment_sum(h, src_r, dst_r, zeros)
    return _final(agg, Wr3.T, hw)


# speedup vs baseline: 1.1048x; 1.1048x over previous
import functools

import jax
import jax.numpy as jnp
from jax import lax
from jax.experimental import pallas as pl
from jax.experimental.pallas import tpu as pltpu
from jax.experimental.pallas import tpu_sc as plsc

N = 10000
E = 320000
D = 128

NC = 2
NS = 16
NW = NC * NS

CH = 80
EPW = E // NW
NCHUNK = EPW // CH
SSTR = 624
LSTR = N - (NS - 1) * SSTR

assert NCHUNK * CH == EPW and NCHUNK % 2 == 1 and (NS - 1) * SSTR + LSTR == N

_mesh = plsc.VectorSubcoreMesh(
    core_axis_name="c", subcore_axis_name="s", num_cores=NC, num_subcores=NS
)


@functools.partial(
    pl.kernel,
    out_type=jax.ShapeDtypeStruct((NC, N, D), jnp.float32),
    mesh=_mesh,
    scratch_types=[
        pltpu.VMEM((EPW,), jnp.int32),
        [pltpu.VMEM((CH,), jnp.int32) for _ in range(3)],
        [pltpu.VMEM((CH, D), jnp.float32) for _ in range(3)],
        pltpu.VMEM_SHARED((N, D), jnp.float32),
        [pltpu.SemaphoreType.DMA for _ in range(3)],
        [pltpu.SemaphoreType.DMA for _ in range(3)],
        [pltpu.SemaphoreType.DMA for _ in range(3)],
    ],
)
def _sc_segment_sum(h_hbm, src_hbm, dst_hbm, zeros_hbm, out_hbm,
                    src_v, dst_bufs, row_bufs, agg_sh, gsems, dsems, ssems):
    c = lax.axis_index("c")
    s = lax.axis_index("s")
    wid = c * NS + s
    ebase = pl.multiple_of(wid * EPW, 8)

    pltpu.sync_copy(src_hbm.at[pl.ds(ebase, EPW)], src_v)

    @pl.when(s < NS - 1)
    def _():
        stripe = pl.ds(pl.multiple_of(s * SSTR, 8), SSTR)
        pltpu.sync_copy(zeros_hbm.at[stripe], agg_sh.at[stripe])

    @pl.when(s == NS - 1)
    def _():
        stripe = pl.ds((NS - 1) * SSTR, LSTR)
        pltpu.sync_copy(zeros_hbm.at[stripe], agg_sh.at[stripe])

    plsc.subcore_barrier()

    def start_g(j, b):
        pltpu.async_copy(h_hbm.at[src_v.at[pl.ds(j * CH, CH)]], row_bufs[b],
                         gsems[b])
        pltpu.async_copy(
            dst_hbm.at[pl.ds(pl.multiple_of(ebase + j * CH, 8), CH)],
            dst_bufs[b], dsems[b])

    def wait_g(b):
        pltpu.make_async_copy(h_hbm.at[src_v.at[pl.ds(0, CH)]], row_bufs[b],
                              gsems[b]).wait()
        pltpu.make_async_copy(dst_hbm.at[pl.ds(0, CH)], dst_bufs[b],
                              dsems[b]).wait()

    def start_s(b):
        pltpu.async_copy(row_bufs[b], agg_sh.at[dst_bufs[b]], ssems[b],
                         add=True)

    def wait_s(b):
        pltpu.make_async_copy(row_bufs[b], agg_sh.at[dst_bufs[b]],
                              ssems[b]).wait()

    start_g(0, 0)
    start_g(1, 1)
    wait_g(0); start_s(0); start_g(2, 2)
    wait_g(1); start_s(1); wait_s(0); start_g(3, 0)
    wait_g(2); start_s(2); wait_s(1); start_g(4, 1)

    @pl.loop(3, NCHUNK - 2, step=3)
    def _(j):
        wait_g(0); start_s(0); wait_s(2); start_g(j + 2, 2)
        wait_g(1); start_s(1); wait_s(0); start_g(j + 3, 0)
        wait_g(2); start_s(2); wait_s(1); start_g(j + 4, 1)

    wait_g(0); start_s(0); wait_s(2)
    wait_g(1); start_s(1); wait_s(0)
    wait_s(1)

    plsc.subcore_barrier()

    @pl.when(s < NS - 1)
    def _():
        stripe = pl.ds(pl.multiple_of(s * SSTR, 8), SSTR)
        pltpu.sync_copy(agg_sh.at[stripe], out_hbm.at[c, stripe])

    @pl.when(s == NS - 1)
    def _():
        stripe = pl.ds((NS - 1) * SSTR, LSTR)
        pltpu.sync_copy(agg_sh.at[stripe], out_hbm.at[c, stripe])


_BLK = 2000
_F32 = jnp.float32


def _dot(a, b):
    return jnp.dot(a, b, preferred_element_type=_F32,
                   precision=lax.Precision.HIGHEST)


def _pre_body(x_ref, wpt_ref, bp_ref, wst_ref, br_ref, xres_ref, hw_ref):
    x = x_ref[...]
    xres_ref[...] = _dot(x, wpt_ref[...]) + bp_ref[...]
    hw_ref[...] = _dot(x, wst_ref[...]) + br_ref[...]


def _pre(x, wpt, bp, wst, br):
    return pl.pallas_call(
        _pre_body,
        out_shape=(jax.ShapeDtypeStruct((N, D), _F32),
                   jax.ShapeDtypeStruct((N, D), _F32)),
        grid=(N // _BLK,),
        in_specs=[
            pl.BlockSpec((_BLK, D), lambda i: (i, 0)),
            pl.BlockSpec((D, D), lambda i: (0, 0)),
            pl.BlockSpec((1, D), lambda i: (0, 0)),
            pl.BlockSpec((D, D), lambda i: (0, 0)),
            pl.BlockSpec((1, D), lambda i: (0, 0)),
        ],
        out_specs=(pl.BlockSpec((_BLK, D), lambda i: (i, 0)),
                   pl.BlockSpec((_BLK, D), lambda i: (i, 0))),
    )(x, wpt, bp.reshape(1, D), wst, br.reshape(1, D))


def _layer_body(agg_a_ref, agg_b_ref, wrt_ref, hw_ref, add_ref,
                wst_ref, brn_ref, h_ref, hwn_ref):
    h = jnp.maximum(_dot(agg_a_ref[0] + agg_b_ref[0], wrt_ref[...])
                    + hw_ref[...], 0.0) + add_ref[...]
    h_ref[...] = h
    hwn_ref[...] = _dot(h, wst_ref[...]) + brn_ref[...]


def _layer(agg, wrt, hw, add, wst_next, br_next):
    return pl.pallas_call(
        _layer_body,
        out_shape=(jax.ShapeDtypeStruct((N, D), _F32),
                   jax.ShapeDtypeStruct((N, D), _F32)),
        grid=(N // _BLK,),
        in_specs=[
            pl.BlockSpec((1, _BLK, D), lambda i: (0, i, 0)),
            pl.BlockSpec((1, _BLK, D), lambda i: (1, i, 0)),
            pl.BlockSpec((D, D), lambda i: (0, 0)),
            pl.BlockSpec((_BLK, D), lambda i: (i, 0)),
            pl.BlockSpec((_BLK, D), lambda i: (i, 0)),
            pl.BlockSpec((D, D), lambda i: (0, 0)),
            pl.BlockSpec((1, D), lambda i: (0, 0)),
        ],
        out_specs=(pl.BlockSpec((_BLK, D), lambda i: (i, 0)),
                   pl.BlockSpec((_BLK, D), lambda i: (i, 0))),
    )(agg, agg, wrt, hw, add, wst_next, br_next.reshape(1, D))


def _final_body(agg_a_ref, agg_b_ref, wrt_ref, hw_ref, o_ref):
    o_ref[...] = _dot(agg_a_ref[0] + agg_b_ref[0], wrt_ref[...]) + hw_ref[...]


def _final(agg, wrt, hw):
    return pl.pallas_call(
        _final_body,
        out_shape=jax.ShapeDtypeStruct((N, D), _F32),
        grid=(N // _BLK,),
        in_specs=[
            pl.BlockSpec((1, _BLK, D), lambda i: (0, i, 0)),
            pl.BlockSpec((1, _BLK, D), lambda i: (1, i, 0)),
            pl.BlockSpec((D, D), lambda i: (0, 0)),
            pl.BlockSpec((_BLK, D), lambda i: (i, 0)),
        ],
        out_specs=pl.BlockSpec((_BLK, D), lambda i: (i, 0)),
    )(agg, agg, wrt, hw)


def kernel(x, edge_index, Wp, bp, Wr0, br0, Ws0, Wr1, br1, Ws1,
           Wr2, br2, Ws2, Wr3, br3, Ws3):
    edge_index = edge_index.astype(jnp.int32)
    src_r = edge_index[0]
    dst_r = edge_index[1]
    zeros = jnp.zeros((N, D), _F32)

    x_res, hw = _pre(x, Wp.T, bp, Ws0.T, br0)

    h = x
    agg = _sc_segment_sum(h, src_r, dst_r, zeros)
    h, hw = _layer(agg, Wr0.T, hw, x_res, Ws1.T, br1)

    agg = _sc_segment_sum(h, src_r, dst_r, zeros)
    h, hw = _layer(agg, Wr1.T, hw, h, Ws2.T, br2)

    agg = _sc_segment_sum(h, src_r, dst_r, zeros)
    h, hw = _layer(agg, Wr2.T, hw, h, Ws3.T, br3)

    agg = _sc_segment_sum(h, src_r, dst_r, zeros)
    return _final(agg, Wr3.T, hw)
